# Initial kernel scaffold; baseline (speedup 1.0000x reference)
import jax
import jax.numpy as jnp
from jax import lax
from jax.experimental import pallas as pl
from jax.experimental.pallas import tpu as pltpu
from jax.experimental.pallas import tpu_sc as plsc

NUM_EMB = 1000000
DIM = 64
RANK = 16
TOTAL = 4096 * 50
NC, NS, LANES = 2, 16, 16
NW = NC * NS
PER_W = TOTAL // NW
CHUNK = 640
NCHUNK = PER_W // CHUNK
SUB = 128
NSUB = CHUNK // SUB


def _sc_kernel(idx_hbm, emb_hbm, a_hbm, bt_hbm, out_hbm,
               idx_v, buf_v, a_v, bt_v, sem):
    wid = lax.axis_index("c") * NS + lax.axis_index("s")

    pltpu.sync_copy(bt_hbm, bt_v)
    btv = [bt_v[pl.ds(k * LANES, LANES)] for k in range(DIM * RANK // LANES)]

    def chunk_body(k, _):
        row0 = wid * PER_W + k * CHUNK
        pltpu.sync_copy(idx_hbm.at[pl.ds(row0 // SUB, NSUB)], idx_v)
        copies = []
        for j in range(NSUB):
            copies.append(pltpu.async_copy(
                emb_hbm.at[idx_v.at[j]], buf_v.at[pl.ds(j * SUB, SUB)], sem))
            copies.append(pltpu.async_copy(
                a_hbm.at[idx_v.at[j]], a_v.at[pl.ds(j * SUB, SUB)], sem))
        for c in copies:
            c.wait()

        def lookup_body(i, _):
            a_s = [a_v[i, r] for r in range(RANK)]
            for c in range(DIM // LANES):
                acc = buf_v[i, pl.ds(c * LANES, LANES)]
                for r in range(RANK):
                    acc = acc + a_s[r] * btv[r * (DIM // LANES) + c]
                buf_v[i, pl.ds(c * LANES, LANES)] = acc
            return 0

        lax.fori_loop(0, CHUNK, lookup_body, 0)
        pltpu.sync_copy(buf_v, out_hbm.at[pl.ds(row0, CHUNK)])
        return 0

    lax.fori_loop(0, NCHUNK, chunk_body, 0)


def kernel(idx, embedding_weight, lora_A_weight, lora_B_weight):
    B, L = idx.shape
    idx2d = idx.reshape(TOTAL // SUB, SUB).astype(jnp.int32)
    bt_flat = lora_B_weight.T.reshape(-1)

    mesh = plsc.VectorSubcoreMesh(core_axis_name="c", subcore_axis_name="s",
                                  num_cores=NC, num_subcores=NS)
    out = pl.kernel(
        _sc_kernel,
        out_type=jax.ShapeDtypeStruct((TOTAL, DIM), jnp.float32),
        mesh=mesh,
        scratch_types=[
            pltpu.VMEM((NSUB, SUB), jnp.int32),
            pltpu.VMEM((CHUNK, DIM), jnp.float32),
            pltpu.VMEM((CHUNK, RANK), jnp.float32),
            pltpu.VMEM((DIM * RANK,), jnp.float32),
            pltpu.SemaphoreType.DMA,
        ],
    )(idx2d, embedding_weight, lora_A_weight, bt_flat)
    return out.reshape(B, L, DIM)


# speedup vs baseline: 3.0692x; 3.0692x over previous
import jax
import jax.numpy as jnp
from jax import lax
from jax.experimental import pallas as pl
from jax.experimental.pallas import tpu as pltpu
from jax.experimental.pallas import tpu_sc as plsc

NUM_EMB = 1000000
DIM = 64
RANK = 16
TOTAL = 4096 * 50
NC, NS, LANES = 2, 16, 16
NW = NC * NS
PER_W = TOTAL // NW
CHUNK = 640
NCHUNK = PER_W // CHUNK
SUB = 128
NSUB = CHUNK // SUB


def _sc_kernel(idx_hbm, emb_hbm, a_hbm, bt_hbm, out_hbm,
               idx_v, buf_v, a_v, bt_v, sem):
    wid = lax.axis_index("c") * NS + lax.axis_index("s")

    pltpu.sync_copy(bt_hbm, bt_v)
    btv = [bt_v[pl.ds(k * LANES, LANES)] for k in range(DIM * RANK // LANES)]

    def chunk_body(k, _):
        row0 = wid * PER_W + k * CHUNK
        pltpu.sync_copy(idx_hbm.at[wid * NCHUNK + k], idx_v)
        copies = []
        for j in range(NSUB):
            copies.append(pltpu.async_copy(
                emb_hbm.at[idx_v.at[j]], buf_v.at[pl.ds(j * SUB, SUB)], sem))
            copies.append(pltpu.async_copy(
                a_hbm.at[idx_v.at[j]], a_v.at[pl.ds(j * SUB, SUB)], sem))
        for c in copies:
            c.wait()

        def lookup_body(i, _):
            a_row = a_v[i]
            a_s = [a_row[r] for r in range(RANK)]
            for c in range(DIM // LANES):
                acc = buf_v[i, pl.ds(c * LANES, LANES)]
                for r in range(RANK):
                    acc = acc + a_s[r] * btv[r * (DIM // LANES) + c]
                buf_v[i, pl.ds(c * LANES, LANES)] = acc
            return 0

        lax.fori_loop(0, CHUNK, lookup_body, 0)
        pltpu.sync_copy(buf_v, out_hbm.at[pl.ds(row0, CHUNK)])
        return 0

    lax.fori_loop(0, NCHUNK, chunk_body, 0)


def kernel(idx, embedding_weight, lora_A_weight, lora_B_weight):
    B, L = idx.shape
    idx3d = idx.reshape(NW * NCHUNK, NSUB, SUB).astype(jnp.int32)
    bt_flat = lora_B_weight.T.reshape(-1)

    mesh = plsc.VectorSubcoreMesh(core_axis_name="c", subcore_axis_name="s",
                                  num_cores=NC, num_subcores=NS)
    out = pl.kernel(
        _sc_kernel,
        out_type=jax.ShapeDtypeStruct((TOTAL, DIM), jnp.float32),
        mesh=mesh,
        compiler_params=pltpu.CompilerParams(use_tc_tiling_on_sc=False),
        scratch_types=[
            pltpu.VMEM((NSUB, SUB), jnp.int32),
            pltpu.VMEM((CHUNK, DIM), jnp.float32),
            pltpu.VMEM((CHUNK, RANK), jnp.float32),
            pltpu.VMEM((DIM * RANK,), jnp.float32),
            pltpu.SemaphoreType.DMA,
        ],
    )(idx3d, embedding_weight, lora_A_weight, bt_flat)
    return out.reshape(B, L, DIM)


# speedup vs baseline: 3.5315x; 1.1506x over previous
import jax
import jax.numpy as jnp
from jax import lax
from jax.experimental import pallas as pl
from jax.experimental.pallas import tpu as pltpu
from jax.experimental.pallas import tpu_sc as plsc

NUM_EMB = 1000000
DIM = 64
RANK = 16
TOTAL = 4096 * 50
NC, NS, LANES = 2, 16, 16
NW = NC * NS
PER_W = TOTAL // NW
CHUNK = 640
NCHUNK = PER_W // CHUNK
SUB = 128
NSUB = CHUNK // SUB


def _sc_kernel(idx_hbm, emb_hbm, a_hbm, bt_hbm, out_hbm,
               idx_v, buf_v, a_v, bt_v, sem):
    wid = lax.axis_index("c") * NS + lax.axis_index("s")

    pltpu.sync_copy(bt_hbm, bt_v)
    btv = [bt_v[pl.ds(k * LANES, LANES)] for k in range(DIM * RANK // LANES)]

    def chunk_body(k, _):
        row0 = wid * PER_W + k * CHUNK
        pltpu.sync_copy(idx_hbm.at[wid * NCHUNK + k], idx_v)
        copies = []
        for j in range(NSUB):
            copies.append(pltpu.async_copy(
                emb_hbm.at[idx_v.at[j]], buf_v.at[pl.ds(j * SUB, SUB)], sem))
            copies.append(pltpu.async_copy(
                a_hbm.at[idx_v.at[j]], a_v.at[pl.ds(j * SUB, SUB)], sem))
        for c in copies:
            c.wait()

        @plsc.parallel_loop(0, CHUNK, unroll=2)
        def lookup_body(i):
            a_row = a_v[i]
            a_s = [a_row[r] for r in range(RANK)]
            for c in range(DIM // LANES):
                acc = buf_v[i, pl.ds(c * LANES, LANES)]
                for r in range(RANK):
                    acc = acc + a_s[r] * btv[r * (DIM // LANES) + c]
                buf_v[i, pl.ds(c * LANES, LANES)] = acc
        pltpu.sync_copy(buf_v, out_hbm.at[pl.ds(row0, CHUNK)])
        return 0

    lax.fori_loop(0, NCHUNK, chunk_body, 0)


def kernel(idx, embedding_weight, lora_A_weight, lora_B_weight):
    B, L = idx.shape
    idx3d = idx.reshape(NW * NCHUNK, NSUB, SUB).astype(jnp.int32)
    bt_flat = lora_B_weight.T.reshape(-1)

    mesh = plsc.VectorSubcoreMesh(core_axis_name="c", subcore_axis_name="s",
                                  num_cores=NC, num_subcores=NS)
    out = pl.kernel(
        _sc_kernel,
        out_type=jax.ShapeDtypeStruct((TOTAL, DIM), jnp.float32),
        mesh=mesh,
        compiler_params=pltpu.CompilerParams(use_tc_tiling_on_sc=False),
        scratch_types=[
            pltpu.VMEM((NSUB, SUB), jnp.int32),
            pltpu.VMEM((CHUNK, DIM), jnp.float32),
            pltpu.VMEM((CHUNK, RANK), jnp.float32),
            pltpu.VMEM((DIM * RANK,), jnp.float32),
            pltpu.SemaphoreType.DMA,
        ],
    )(idx3d, embedding_weight, lora_A_weight, bt_flat)
    return out.reshape(B, L, DIM)
